# Initial kernel scaffold; baseline (speedup 1.0000x reference)
import jax
import jax.numpy as jnp
from jax.experimental import pallas as pl


def kernel(x, u, W, b, bank, n_copies):
    raise NotImplementedError("write your pallas kernel here")


# speedup vs baseline: 3.4431x; 3.4431x over previous
import functools

import jax
import jax.numpy as jnp
from jax import lax
from jax.experimental import pallas as pl
from jax.experimental.pallas import tpu as pltpu
from jax.experimental.pallas import tpu_sc as plsc

N_CAT = 238
D_IMG = 3 * 32 * 32
BM = 512


def _head_body(x_ref, w_ref, b_ref, u_ref, samp_ref, slp_ref, ent_ref, kl_ref):
    n_copies = u_ref.shape[0]
    x = x_ref[...]
    w = w_ref[...]
    logits = jnp.dot(x, w, preferred_element_type=jnp.float32) + b_ref[...]
    m = jnp.max(logits, axis=-1, keepdims=True)
    sh = logits - m
    lse = jnp.log(jnp.sum(jnp.exp(sh), axis=-1, keepdims=True))
    logp = sh - lse
    p = jnp.exp(logp)
    ent_ref[0, :] = -jnp.sum(p * logp, axis=-1)
    kl_ref[0, :] = jnp.sum(p * (logp - jnp.log(1.0 / N_CAT)), axis=-1)
    iota = lax.broadcasted_iota(jnp.int32, (BM, N_CAT), 1)
    for k in range(n_copies):
        g = -jnp.log(-jnp.log(u_ref[k]))
        s = jnp.argmax(logp + g, axis=-1).astype(jnp.int32)
        samp_ref[k, :] = s
        slp_ref[k, :] = jnp.sum(jnp.where(iota == s[:, None], logp, 0.0), axis=-1)


def _head_call(xf, W, b2, u):
    bsz = xf.shape[0]
    n_copies = u.shape[0]
    grid = (bsz // BM,)
    return pl.pallas_call(
        _head_body,
        grid=grid,
        in_specs=[
            pl.BlockSpec((BM, D_IMG), lambda i: (i, 0)),
            pl.BlockSpec((D_IMG, N_CAT), lambda i: (0, 0)),
            pl.BlockSpec((1, N_CAT), lambda i: (0, 0)),
            pl.BlockSpec((n_copies, BM, N_CAT), lambda i: (0, i, 0)),
        ],
        out_specs=[
            pl.BlockSpec((n_copies, BM), lambda i: (0, i)),
            pl.BlockSpec((n_copies, BM), lambda i: (0, i)),
            pl.BlockSpec((1, BM), lambda i: (0, i)),
            pl.BlockSpec((1, BM), lambda i: (0, i)),
        ],
        out_shape=[
            jax.ShapeDtypeStruct((n_copies, bsz), jnp.int32),
            jax.ShapeDtypeStruct((n_copies, bsz), jnp.float32),
            jax.ShapeDtypeStruct((1, bsz), jnp.float32),
            jax.ShapeDtypeStruct((1, bsz), jnp.float32),
        ],
        compiler_params=pltpu.CompilerParams(
            dimension_semantics=("parallel",),
        ),
    )(xf, W, b2, u)


_SC_CH = 32


def _sc_gather_body(per_w, idx_hbm, bank_hbm, out_hbm, idx_v, rows_v, sem):
    nc = 2
    wid = lax.axis_index("s") * nc + lax.axis_index("c")
    base = wid * per_w
    pltpu.sync_copy(idx_hbm.at[pl.ds(base, per_w)], idx_v)
    for c in range(per_w // _SC_CH):
        off = c * _SC_CH
        pltpu.async_copy(
            bank_hbm.at[idx_v.at[pl.ds(off, _SC_CH)]], rows_v, sem
        ).wait()
        pltpu.sync_copy(rows_v, out_hbm.at[pl.ds(base + off, _SC_CH)])


def _gather_call(idx, bank_flat):
    n_rows = idx.shape[0]
    info = plsc.get_sparse_core_info()
    nw = info.num_cores * info.num_subcores
    per_w = n_rows // nw
    mesh = plsc.VectorSubcoreMesh(core_axis_name="c", subcore_axis_name="s")
    kfn = pl.kernel(
        functools.partial(_sc_gather_body, per_w),
        mesh=mesh,
        out_type=jax.ShapeDtypeStruct((n_rows, D_IMG), jnp.float32),
        scratch_types=[
            pltpu.VMEM((per_w,), jnp.int32),
            pltpu.VMEM((_SC_CH, D_IMG), jnp.float32),
            pltpu.SemaphoreType.DMA,
        ],
    )
    return kfn(idx, bank_flat)


def kernel(x, u, W, b, bank, n_copies):
    bsz = x.shape[0]
    n_copies_static = u.shape[0]
    xf = x.reshape(bsz, -1)
    samp, slp, ent, kl = _head_call(xf, W, b.reshape(1, -1), u)
    idx = samp.reshape(-1)
    bank_flat = bank.reshape(N_CAT, D_IMG)
    rows = _gather_call(idx, bank_flat)
    x_out = jax.lax.stop_gradient(
        rows.reshape(n_copies_static * bsz, *bank.shape[1:])
    )
    return (x_out, slp.reshape(-1), ent.reshape(-1), kl.reshape(-1))
